# baseline (speedup 1.0000x reference)
import dataclasses

import jax
import jax.numpy as jnp
from jax import lax
from jax.experimental import pallas as pl
from jax.experimental.pallas import tpu as pltpu
from jax.experimental.pallas import tpu_sc as plsc

V = 1000
VP = 1024
N = 20480
NH = N // 2
NC, NS, L = 2, 16, 16
NW = NC * NS
BPW = NH // NW
C = 32
ITERS = BPW // C
G = C // L


def _lse_body(t_ref, o_ref):
    t = t_ref[...]
    m = jnp.max(t, axis=1)
    s = jnp.sum(jnp.exp(t - m[:, None]), axis=1)
    o_ref[...] = m + jnp.log(s)


def _loss_body(p_ref, o_ref):
    o_ref[...] = jnp.reshape(jnp.sum(p_ref[...]) / N, (1, 1))


TR = 1024


def _tpose_body(t_ref, o_ref):
    o_ref[...] = t_ref[...].T[:V, :]


def _sc_body(table_hbm, idx_hbm, tgt_hbm, lse_hbm,
             logits_hbm, part_hbm,
             idx_v, tgt_v, lse_v, buf0, buf1, buf2, acc_v,
             sem0, sem1, sem2, wsem0, wsem1, wsem2):
    wid = lax.axis_index("c") * NS + lax.axis_index("s")
    base = wid * BPW

    pltpu.sync_copy(idx_hbm.at[pl.ds(base, BPW)], idx_v)
    pltpu.sync_copy(tgt_hbm.at[pl.ds(base, BPW)], tgt_v)
    pltpu.sync_copy(lse_hbm, lse_v)
    acc_v[...] = jnp.zeros((L,), jnp.float32)

    NB = 3
    bufs = (buf0, buf1, buf2)
    sems = (sem0, sem1, sem2)
    wsems = (wsem0, wsem1, wsem2)

    def start(g):
        return pltpu.async_copy(
            table_hbm.at[idx_v.at[pl.ds(g * C, C)]], bufs[g % NB], sems[g % NB])

    pend_g = [None] * ITERS
    pend_w = [None] * NB
    pend_g[0] = start(0)
    pend_g[1] = start(1)
    for g in range(ITERS):
        pend_g[g].wait()
        if g + 2 < ITERS:
            if pend_w[(g + 2) % NB] is not None:
                pend_w[(g + 2) % NB].wait()
            pend_g[g + 2] = start(g + 2)
        buf = bufs[g % NB]
        for g2 in range(G):
            off = g * C + g2 * L
            rows = lax.iota(jnp.int32, L) + (g2 * L)
            cols = tgt_v.at[pl.ds(off, L)][...]
            tvals = plsc.load_gather(buf, [rows, cols])
            lvals = plsc.load_gather(lse_v, [idx_v.at[pl.ds(off, L)][...]])
            acc_v[...] = acc_v[...] + (lvals - tvals)
        pend_w[g % NB] = pltpu.async_copy(
            buf, logits_hbm.at[pl.ds(base + g * C, C)], wsems[g % NB])

    for b in range(NB):
        if pend_w[b] is not None:
            pend_w[b].wait()
    pltpu.sync_copy(acc_v, part_hbm.at[wid])


@jax.jit
def kernel(index, targets, embedding_table):
    idx_flat = index.reshape(-1)
    tgt_flat = targets.reshape(-1)
    table_pad = jnp.pad(embedding_table, ((0, 0), (0, VP - V)))

    lse = pl.pallas_call(
        _lse_body,
        out_shape=jax.ShapeDtypeStruct((V,), jnp.float32),
    )(embedding_table)

    cp = pltpu.CompilerParams()
    if "needs_layout_passes" in pltpu.CompilerParams.__dataclass_fields__:
        cp = dataclasses.replace(cp, needs_layout_passes=False)
    mesh = plsc.VectorSubcoreMesh(core_axis_name="c", subcore_axis_name="s")
    sc = pl.kernel(
        _sc_body,
        mesh=mesh,
        compiler_params=cp,
        out_type=(
            jax.ShapeDtypeStruct((NH, VP), jnp.float32),
            jax.ShapeDtypeStruct((NW, L), jnp.float32),
        ),
        scratch_types=[
            pltpu.VMEM((BPW,), jnp.int32),
            pltpu.VMEM((BPW,), jnp.int32),
            pltpu.VMEM((V,), jnp.float32),
            pltpu.VMEM((C, VP), jnp.float32),
            pltpu.VMEM((C, VP), jnp.float32),
            pltpu.VMEM((C, VP), jnp.float32),
            pltpu.VMEM((L,), jnp.float32),
            pltpu.SemaphoreType.DMA,
            pltpu.SemaphoreType.DMA,
            pltpu.SemaphoreType.DMA,
            pltpu.SemaphoreType.DMA,
            pltpu.SemaphoreType.DMA,
            pltpu.SemaphoreType.DMA,
        ],
    )
    logits_a, part_a = sc(table_pad, idx_flat[:NH], tgt_flat[:NH], lse)
    logits_b, part_b = sc(table_pad, idx_flat[NH:], tgt_flat[NH:], lse)

    tpose = pl.pallas_call(
        _tpose_body,
        grid=(NH // TR,),
        in_specs=[pl.BlockSpec((TR, VP), lambda i: (i, 0))],
        out_specs=pl.BlockSpec((V, TR), lambda i: (0, i)),
        out_shape=jax.ShapeDtypeStruct((V, NH), jnp.float32),
    )
    logits_t = jnp.concatenate([tpose(logits_a), tpose(logits_b)], axis=1)
    logits2d = logits_t.T

    loss2d = pl.pallas_call(
        _loss_body,
        out_shape=jax.ShapeDtypeStruct((1, 1), jnp.float32),
    )(jnp.concatenate([part_a, part_b], axis=0))

    return logits2d, loss2d[0, 0]


# speedup vs baseline: 1.3302x; 1.3302x over previous
import dataclasses

import jax
import jax.numpy as jnp
from jax import lax
from jax.experimental import pallas as pl
from jax.experimental.pallas import tpu as pltpu
from jax.experimental.pallas import tpu_sc as plsc

V = 1000
VP = 1024
N = 20480
NC, NS, L = 2, 16, 16
NW = NC * NS
BPW = N // NW
C = 32
ITERS = BPW // C
G = C // L


def _lse_body(t_ref, o_ref):
    t = t_ref[...]
    m = jnp.max(t, axis=1)
    s = jnp.sum(jnp.exp(t - m[:, None]), axis=1)
    o_ref[...] = m + jnp.log(s)


def _loss_body(p_ref, o_ref):
    o_ref[...] = jnp.reshape(jnp.sum(p_ref[...]) / N, (1, 1))


TR = 1024


def _tpose_body(t_ref, o_ref):
    o_ref[...] = t_ref[...].T[:V, :]


def _sc_body(table_hbm, idx_hbm, tgt_hbm, lse_hbm,
             logits_hbm, part_hbm,
             idx_v, tgt_v, lse_v, buf0, buf1, buf2, acc_v,
             sem0, sem1, sem2, wsem0, wsem1, wsem2):
    wid = lax.axis_index("c") * NS + lax.axis_index("s")
    base = wid * BPW

    pltpu.sync_copy(idx_hbm.at[pl.ds(base, BPW)], idx_v)
    pltpu.sync_copy(tgt_hbm.at[pl.ds(base, BPW)], tgt_v)
    pltpu.sync_copy(lse_hbm, lse_v)
    acc_v[...] = jnp.zeros((L,), jnp.float32)

    NB = 3
    bufs = (buf0, buf1, buf2)
    sems = (sem0, sem1, sem2)
    wsems = (wsem0, wsem1, wsem2)

    def start(g):
        return pltpu.async_copy(
            table_hbm.at[idx_v.at[pl.ds(g * C, C)]], bufs[g % NB], sems[g % NB])

    pend_g = [None] * ITERS
    pend_w = [None] * NB
    pend_g[0] = start(0)
    pend_g[1] = start(1)
    for g in range(ITERS):
        pend_g[g].wait()
        if g + 2 < ITERS:
            if pend_w[(g + 2) % NB] is not None:
                pend_w[(g + 2) % NB].wait()
            pend_g[g + 2] = start(g + 2)
        buf = bufs[g % NB]
        for g2 in range(G):
            off = g * C + g2 * L
            rows = lax.iota(jnp.int32, L) + (g2 * L)
            cols = tgt_v.at[pl.ds(off, L)][...]
            tvals = plsc.load_gather(buf, [rows, cols])
            lvals = plsc.load_gather(lse_v, [idx_v.at[pl.ds(off, L)][...]])
            acc_v[...] = acc_v[...] + (lvals - tvals)
        pend_w[g % NB] = pltpu.async_copy(
            buf, logits_hbm.at[pl.ds(base + g * C, C)], wsems[g % NB])

    for b in range(NB):
        if pend_w[b] is not None:
            pend_w[b].wait()
    pltpu.sync_copy(acc_v, part_hbm.at[wid])


@jax.jit
def kernel(index, targets, embedding_table):
    idx_flat = index.reshape(-1)
    tgt_flat = targets.reshape(-1)
    table_pad = jnp.pad(embedding_table, ((0, 0), (0, VP - V)))

    lse = pl.pallas_call(
        _lse_body,
        out_shape=jax.ShapeDtypeStruct((V,), jnp.float32),
    )(embedding_table)

    cp = pltpu.CompilerParams()
    if "needs_layout_passes" in pltpu.CompilerParams.__dataclass_fields__:
        cp = dataclasses.replace(cp, needs_layout_passes=False)
    mesh = plsc.VectorSubcoreMesh(core_axis_name="c", subcore_axis_name="s")
    sc = pl.kernel(
        _sc_body,
        mesh=mesh,
        compiler_params=cp,
        out_type=(
            jax.ShapeDtypeStruct((N, VP), jnp.float32),
            jax.ShapeDtypeStruct((NW, L), jnp.float32),
        ),
        scratch_types=[
            pltpu.VMEM((BPW,), jnp.int32),
            pltpu.VMEM((BPW,), jnp.int32),
            pltpu.VMEM((V,), jnp.float32),
            pltpu.VMEM((C, VP), jnp.float32),
            pltpu.VMEM((C, VP), jnp.float32),
            pltpu.VMEM((C, VP), jnp.float32),
            pltpu.VMEM((L,), jnp.float32),
            pltpu.SemaphoreType.DMA,
            pltpu.SemaphoreType.DMA,
            pltpu.SemaphoreType.DMA,
            pltpu.SemaphoreType.DMA,
            pltpu.SemaphoreType.DMA,
            pltpu.SemaphoreType.DMA,
        ],
    )
    logits_pad, partials = sc(table_pad, idx_flat, tgt_flat, lse)

    logits_t = pl.pallas_call(
        _tpose_body,
        grid=(N // TR,),
        in_specs=[pl.BlockSpec((TR, VP), lambda i: (i, 0))],
        out_specs=pl.BlockSpec((V, TR), lambda i: (0, i)),
        out_shape=jax.ShapeDtypeStruct((V, N), jnp.float32),
    )(logits_pad)
    logits2d = logits_t.T

    loss2d = pl.pallas_call(
        _loss_body,
        out_shape=jax.ShapeDtypeStruct((1, 1), jnp.float32),
    )(partials)

    return logits2d, loss2d[0, 0]
